# Initial kernel scaffold; baseline (speedup 1.0000x reference)
import jax
import jax.numpy as jnp
from jax.experimental import pallas as pl


def kernel(x, conv1_w, conv1_b, bn0_g, bn0_b, sa_w, sa_b, gcn_w, gcn_b, bn_g, bn_b, fc1_w, fc1_b, fc2_w, fc2_b):
    raise NotImplementedError("write your pallas kernel here")


# speedup vs baseline: 4.3414x; 4.3414x over previous
import jax
import jax.numpy as jnp
from jax.experimental import pallas as pl
from jax.experimental.pallas import tpu as pltpu

_HID = 64
_H = 224
_W = 224
_B = 2
_RT = 16
_TPI = _H // _RT
_NT = _B * _TPI
_PIX = _B * _H * _W
_EPS = 1e-5


def _stats_kernel(x_ref, sum_ref, gram_ref):
    t = pl.program_id(0)

    @pl.when(t == 0)
    def _():
        sum_ref[...] = jnp.zeros_like(sum_ref)
        gram_ref[...] = jnp.zeros_like(gram_ref)

    xb = x_ref[0]
    sum_ref[...] += jnp.sum(xb, axis=0, keepdims=True)
    gram_ref[...] += jax.lax.dot_general(
        xb, xb, (((0,), (0,)), ((), ())), preferred_element_type=jnp.float32)


def _feat_kernel(x_ref, w_ref, b_ref, f_ref):
    xb = x_ref[0]
    f = jnp.maximum(
        jax.lax.dot_general(xb, w_ref[...], (((2,), (1,)), ((), ())),
                            preferred_element_type=jnp.float32) + b_ref[...],
        0.0)
    f_ref[0, :, 0:3, :] = jnp.zeros((_RT, 3, _HID), jnp.float32)
    f_ref[0, :, 3:3 + _W, :] = f
    f_ref[0, :, 3 + _W:6 + _W, :] = jnp.zeros((_RT, 3, _HID), jnp.float32)


def _att_gcn0_kernel(fm_ref, fc_ref, fp_ref, wsa_ref, sab_ref, w0_ref,
                     out_ref, stk_ref):
    t = pl.program_id(0)
    y0 = (t % _TPI) * _RT
    stk_ref[0:_RT] = fm_ref[0]
    stk_ref[_RT:2 * _RT] = fc_ref[0]
    stk_ref[2 * _RT:3 * _RT] = fp_ref[0]
    rows = jax.lax.broadcasted_iota(jnp.int32, (_RT, 1, 1), 0) + y0

    def tap(i, acc):
        ky = i // 7
        kx = i - ky * 7
        patch = stk_ref[pl.ds(_RT - 3 + ky, _RT), pl.ds(kx, _W), :]
        wv = wsa_ref[pl.ds(i, 1), :]
        contrib = jax.lax.dot_general(
            patch, wv, (((2,), (1,)), ((), ())),
            preferred_element_type=jnp.float32)
        ysrc = rows + ky - 3
        vmask = (ysrc >= 0) & (ysrc < _H)
        return acc + jnp.where(vmask, contrib, 0.0)

    acc = jax.lax.fori_loop(0, 49, tap, jnp.zeros((_RT, _W, 1), jnp.float32))
    att = jax.nn.sigmoid(acc + sab_ref[0, 0])
    g = fc_ref[0, :, 3:3 + _W, :] * att
    out_ref[0] = jax.lax.dot_general(
        g, w0_ref[...], (((2,), (1,)), ((), ())),
        preferred_element_type=jnp.float32)


def _agg_kernel(xm_ref, xc_ref, xp_ref, bias_ref, agg_ref, s_ref, ss_ref):
    t = pl.program_id(0)
    y0 = (t % _TPI) * _RT
    img0 = t < _TPI
    xl = xc_ref[0]

    @pl.when(t == 0)
    def _():
        s_ref[...] = jnp.zeros_like(s_ref)
        ss_ref[...] = jnp.zeros_like(ss_ref)

    ext = jnp.concatenate([xm_ref[0, _RT - 1:_RT], xl, xp_ref[0, 0:1]], axis=0)
    yy_ext = (jax.lax.broadcasted_iota(jnp.int32, (_RT + 2, _W), 0) + y0 - 1)
    xx = jax.lax.broadcasted_iota(jnp.int32, (_RT + 2, _W), 1)
    deg = (1.0 + (yy_ext > 0).astype(jnp.float32)
           + (yy_ext < _H - 1).astype(jnp.float32)
           + (xx > 0).astype(jnp.float32)
           + (xx < _W - 1).astype(jnp.float32))
    dis_ext = jax.lax.rsqrt(deg)[:, :, None]
    u_ext = dis_ext * ext
    u = u_ext[1:_RT + 1]
    up = u_ext[0:_RT]
    down = u_ext[2:_RT + 2]
    yy = yy_ext[1:_RT + 1, 0:1]
    up = jnp.where((yy > 0)[:, :, None], up, 0.0)
    down = jnp.where((yy < _H - 1)[:, :, None], down, 0.0)
    zcol = jnp.zeros((_RT, 1, _HID), jnp.float32)
    left = jnp.concatenate([zcol, u[:, :_W - 1]], axis=1)
    right = jnp.concatenate([u[:, 1:], zcol], axis=1)
    s4 = up + down + left + right
    dis_c = dis_ext[1:_RT + 1]
    agg0 = dis_c * (s4 + u)
    agg = jnp.where(img0, agg0, xl) + bias_ref[...]
    agg_ref[0] = agg
    s_ref[...] += jnp.sum(agg, axis=(0, 1)).reshape(1, _HID)
    ss_ref[...] += jnp.sum(agg * agg, axis=(0, 1)).reshape(1, _HID)


def _bn_matmul_kernel(a_ref, sc_ref, sh_ref, w_ref, out_ref):
    hf = jnp.maximum(a_ref[0] * sc_ref[...] + sh_ref[...], 0.0)
    out_ref[0] = jax.lax.dot_general(
        hf, w_ref[...], (((2,), (1,)), ((), ())),
        preferred_element_type=jnp.float32)


def _pool_kernel(a_ref, sc_ref, sh_ref, out_ref):
    t = pl.program_id(0)

    @pl.when(t == 0)
    def _():
        out_ref[...] = jnp.zeros_like(out_ref)

    hf = jnp.maximum(a_ref[0] * sc_ref[...] + sh_ref[...], 0.0)
    srow = jnp.sum(hf, axis=(0, 1)).reshape(1, _HID)
    img = t // _TPI
    sel = jax.lax.broadcasted_iota(jnp.int32, (_B, 1), 0) == img
    out_ref[...] += jnp.where(sel, srow, 0.0)


def _mlp_kernel(p_ref, w1_ref, b1_ref, w2_ref, b2_ref, o_ref):
    p = p_ref[...]
    h = jnp.maximum(
        jax.lax.dot_general(p, w1_ref[...], (((1,), (1,)), ((), ())),
                            preferred_element_type=jnp.float32) + b1_ref[...],
        0.0)
    o_ref[...] = jax.lax.dot_general(
        h, w2_ref[...], (((1,), (1,)), ((), ())),
        preferred_element_type=jnp.float32) + b2_ref[...]


def _spec(shape, imap):
    return pl.BlockSpec(shape, imap)


def kernel(x, conv1_w, conv1_b, bn0_g, bn0_b, sa_w, sa_b, gcn_w, gcn_b,
           bn_g, bn_b, fc1_w, fc1_b, fc2_w, fc2_b):
    xt = x.transpose(0, 2, 3, 1)
    x3 = xt.reshape(_NT, _RT * _W, _HID)
    x4 = xt.reshape(_NT, _RT, _W, _HID)

    sumx, gram = pl.pallas_call(
        _stats_kernel,
        grid=(_NT,),
        in_specs=[_spec((1, _RT * _W, _HID), lambda t: (t, 0, 0))],
        out_specs=[_spec((1, _HID), lambda t: (0, 0)),
                   _spec((_HID, _HID), lambda t: (0, 0))],
        out_shape=[jax.ShapeDtypeStruct((1, _HID), jnp.float32),
                   jax.ShapeDtypeStruct((_HID, _HID), jnp.float32)],
    )(x3)

    npix = jnp.float32(_PIX)
    mu_x = sumx / npix
    cov = gram / npix - mu_x.T @ mu_x
    w1 = conv1_w.reshape(_HID, _HID)
    var_y = jnp.sum((w1 @ cov) * w1, axis=1)
    mu_y = w1 @ mu_x[0] + conv1_b
    s0 = bn0_g / jnp.sqrt(var_y + _EPS)
    w1p = w1 * s0[:, None]
    b1p = (conv1_b - mu_y) * s0 + bn0_b

    wsa = sa_w[0].transpose(1, 2, 0).reshape(49, _HID)

    f = pl.pallas_call(
        _feat_kernel,
        grid=(_NT,),
        in_specs=[_spec((1, _RT, _W, _HID), lambda t: (t, 0, 0, 0)),
                  _spec((_HID, _HID), lambda t: (0, 0)),
                  _spec((1, _HID), lambda t: (0, 0))],
        out_specs=_spec((1, _RT, _W + 6, _HID), lambda t: (t, 0, 0, 0)),
        out_shape=jax.ShapeDtypeStruct((_NT, _RT, _W + 6, _HID), jnp.float32),
    )(x4, w1p, b1p.reshape(1, _HID))

    cur = lambda t: (t, 0, 0, 0)
    prv = lambda t: (jnp.maximum(t - 1, 0), 0, 0, 0)
    nxt = lambda t: (jnp.minimum(t + 1, _NT - 1), 0, 0, 0)
    fpad = _spec((1, _RT, _W + 6, _HID), cur)
    xl = pl.pallas_call(
        _att_gcn0_kernel,
        grid=(_NT,),
        in_specs=[_spec((1, _RT, _W + 6, _HID), prv),
                  fpad,
                  _spec((1, _RT, _W + 6, _HID), nxt),
                  _spec((49, _HID), lambda t: (0, 0)),
                  _spec((1, 1), lambda t: (0, 0)),
                  _spec((_HID, _HID), lambda t: (0, 0))],
        out_specs=_spec((1, _RT, _W, _HID), cur),
        out_shape=jax.ShapeDtypeStruct((_NT, _RT, _W, _HID), jnp.float32),
        scratch_shapes=[pltpu.VMEM((3 * _RT, _W + 6, _HID), jnp.float32)],
    )(f, f, f, wsa, sa_b.reshape(1, 1), gcn_w[0])

    scale = None
    shift = None
    for l in range(gcn_w.shape[0]):
        if l > 0:
            xl = pl.pallas_call(
                _bn_matmul_kernel,
                grid=(_NT,),
                in_specs=[_spec((1, _RT, _W, _HID), cur),
                          _spec((1, _HID), lambda t: (0, 0)),
                          _spec((1, _HID), lambda t: (0, 0)),
                          _spec((_HID, _HID), lambda t: (0, 0))],
                out_specs=_spec((1, _RT, _W, _HID), cur),
                out_shape=jax.ShapeDtypeStruct((_NT, _RT, _W, _HID),
                                               jnp.float32),
            )(agg, scale.reshape(1, _HID), shift.reshape(1, _HID), gcn_w[l])
        agg, s1, s2 = pl.pallas_call(
            _agg_kernel,
            grid=(_NT,),
            in_specs=[_spec((1, _RT, _W, _HID), prv),
                      _spec((1, _RT, _W, _HID), cur),
                      _spec((1, _RT, _W, _HID), nxt),
                      _spec((1, _HID), lambda t: (0, 0))],
            out_specs=[_spec((1, _RT, _W, _HID), cur),
                       _spec((1, _HID), lambda t: (0, 0)),
                       _spec((1, _HID), lambda t: (0, 0))],
            out_shape=[jax.ShapeDtypeStruct((_NT, _RT, _W, _HID), jnp.float32),
                       jax.ShapeDtypeStruct((1, _HID), jnp.float32),
                       jax.ShapeDtypeStruct((1, _HID), jnp.float32)],
        )(xl, xl, xl, gcn_b[l].reshape(1, _HID))
        mean = s1[0] / npix
        var = s2[0] / npix - mean * mean
        scale = bn_g[l] / jnp.sqrt(var + _EPS)
        shift = bn_b[l] - mean * scale

    pooled_sums = pl.pallas_call(
        _pool_kernel,
        grid=(_NT,),
        in_specs=[_spec((1, _RT, _W, _HID), cur),
                  _spec((1, _HID), lambda t: (0, 0)),
                  _spec((1, _HID), lambda t: (0, 0))],
        out_specs=_spec((_B, _HID), lambda t: (0, 0)),
        out_shape=jax.ShapeDtypeStruct((_B, _HID), jnp.float32),
    )(agg, scale.reshape(1, _HID), shift.reshape(1, _HID))
    pooled = pooled_sums / jnp.float32(_H * _W)

    out = pl.pallas_call(
        _mlp_kernel,
        in_specs=[pl.BlockSpec((_B, _HID), lambda: (0, 0)),
                  pl.BlockSpec((_HID, _HID), lambda: (0, 0)),
                  pl.BlockSpec((1, _HID), lambda: (0, 0)),
                  pl.BlockSpec((16, _HID), lambda: (0, 0)),
                  pl.BlockSpec((1, 16), lambda: (0, 0))],
        out_specs=pl.BlockSpec((_B, 16), lambda: (0, 0)),
        out_shape=jax.ShapeDtypeStruct((_B, 16), jnp.float32),
    )(pooled, fc1_w, fc1_b.reshape(1, _HID), fc2_w, fc2_b.reshape(1, 16))
    return out
